# baseline (speedup 1.0000x reference)
import functools

import jax
import jax.numpy as jnp
from jax.experimental import pallas as pl
from jax.experimental.pallas import tpu as pltpu

U_TILE = 512
L_CHUNK = 1024
LANES = 128
K_AUG = 80
NBUF = 3


def _main_kernel(pred_hbm, u_ref, l_ref, uraw_ref, d2min_ref,
                 laug_ref, pbuf, psem):
    i = pl.program_id(0)
    n = pl.num_programs(0)
    n_l = l_ref.shape[0]
    n_feat = l_ref.shape[1]

    def pcopy(k):
        return pltpu.make_async_copy(
            pred_hbm.at[pl.ds(k * U_TILE, U_TILE), :],
            pbuf.at[jax.lax.rem(k, NBUF)],
            psem.at[jax.lax.rem(k, NBUF)])

    @pl.when(i == 0)
    def _prologue():
        pcopy(0).start()
        pcopy(1).start()
        lz = l_ref[...]
        lh = -0.5 * jnp.sum(lz * lz, axis=1, keepdims=True)
        lh_hi = lh.astype(jnp.bfloat16)
        lh_lo = (lh - lh_hi.astype(jnp.float32)).astype(jnp.bfloat16)
        pad = jnp.zeros((n_l, K_AUG - n_feat - 2), jnp.bfloat16)
        laug_ref[...] = jnp.concatenate(
            [lz.astype(jnp.bfloat16), lh_hi, lh_lo, pad], axis=1)

    @pl.when(i + 2 < n)
    def _prefetch():
        pcopy(i + 2).start()

    uq = u_ref[...]
    u_sq = jnp.sum(uq * uq, axis=1)
    uq_aug = jnp.concatenate(
        [uq.astype(jnp.bfloat16),
         jnp.ones((U_TILE, 2), jnp.bfloat16),
         jnp.zeros((U_TILE, K_AUG - n_feat - 2), jnp.bfloat16)], axis=1)

    carry = jnp.full((U_TILE, LANES), -jnp.inf, dtype=jnp.float32)
    for k in range(n_l // L_CHUNK):
        lc = laug_ref[pl.ds(k * L_CHUNK, L_CHUNK), :]
        t = jax.lax.dot_general(
            uq_aug, lc, (((1,), (1,)), ((), ())),
            preferred_element_type=jnp.float32)
        ms = [jnp.maximum(t[:, (2 * a) * LANES:(2 * a + 1) * LANES],
                          t[:, (2 * a + 1) * LANES:(2 * a + 2) * LANES])
              for a in range(L_CHUNK // (2 * LANES))]
        while len(ms) > 1:
            ms = [jnp.maximum(ms[b], ms[b + 1]) for b in range(0, len(ms), 2)]
        carry = jnp.maximum(carry, ms[0])
    d2min_ref[...] = u_sq - 2.0 * jnp.max(carry, axis=1)

    pcopy(i).wait()
    x = pbuf[jax.lax.rem(i, NBUF)]
    uraw_ref[...] = -jnp.sum(jnp.exp(x) * x, axis=1)


def _finalize_kernel(uraw_ref, d2min_ref, lam_ref, out_ref):
    u = uraw_ref[...]
    u = u - jnp.min(u)
    u = u / (jnp.max(u) + 1e-18)
    d = jnp.sqrt(jnp.maximum(d2min_ref[...], 0.0))
    d = d - jnp.min(d)
    d = d / (jnp.max(d) + 1e-18)
    out_ref[...] = lam_ref[0] * u + d


@functools.partial(jax.jit, static_argnames=("interpret",))
def kernel(pred, U_z, L_z, lambda_, interpret=False):
    n_u = U_z.shape[0]
    n_l = L_z.shape[0]
    grid = (n_u // U_TILE,)
    uraw, d2min = pl.pallas_call(
        _main_kernel,
        grid=grid,
        in_specs=[
            pl.BlockSpec(memory_space=pl.ANY),
            pl.BlockSpec((U_TILE, U_z.shape[1]), lambda i: (i, 0)),
            pl.BlockSpec((n_l, L_z.shape[1]), lambda i: (0, 0)),
        ],
        out_specs=[
            pl.BlockSpec((U_TILE,), lambda i: (i,)),
            pl.BlockSpec((U_TILE,), lambda i: (i,)),
        ],
        out_shape=[
            jax.ShapeDtypeStruct((n_u,), jnp.float32),
            jax.ShapeDtypeStruct((n_u,), jnp.float32),
        ],
        scratch_shapes=[
            pltpu.VMEM((n_l, K_AUG), jnp.bfloat16),
            pltpu.VMEM((NBUF, U_TILE, pred.shape[1]), jnp.float32),
            pltpu.SemaphoreType.DMA((NBUF,)),
        ],
        interpret=interpret,
    )(pred, U_z, L_z)

    lam = jnp.asarray(lambda_, jnp.float32).reshape((1,))
    out = pl.pallas_call(
        _finalize_kernel,
        in_specs=[
            pl.BlockSpec((n_u,), lambda: (0,)),
            pl.BlockSpec((n_u,), lambda: (0,)),
            pl.BlockSpec(memory_space=pltpu.SMEM),
        ],
        out_shape=jax.ShapeDtypeStruct((n_u,), jnp.float32),
        interpret=interpret,
    )(uraw, d2min, lam)
    return out


# speedup vs baseline: 1.0280x; 1.0280x over previous
import functools

import jax
import jax.numpy as jnp
from jax.experimental import pallas as pl
from jax.experimental.pallas import tpu as pltpu

U_TILE = 256
L_CHUNK = 1024
LANES = 128
K_AUG = 80
NBUF = 3


def _main_kernel(pred_hbm, u_ref, l_ref, uraw_ref, d2min_ref,
                 laug_ref, pbuf, psem):
    i = pl.program_id(0)
    n = pl.num_programs(0)
    n_l = l_ref.shape[0]
    n_feat = l_ref.shape[1]

    def pcopy(k):
        return pltpu.make_async_copy(
            pred_hbm.at[pl.ds(k * U_TILE, U_TILE), :],
            pbuf.at[jax.lax.rem(k, NBUF)],
            psem.at[jax.lax.rem(k, NBUF)])

    @pl.when(i == 0)
    def _prologue():
        pcopy(0).start()
        pcopy(1).start()
        lz = l_ref[...]
        lh = -0.5 * jnp.sum(lz * lz, axis=1, keepdims=True)
        lh_hi = lh.astype(jnp.bfloat16)
        lh_lo = (lh - lh_hi.astype(jnp.float32)).astype(jnp.bfloat16)
        pad = jnp.zeros((n_l, K_AUG - n_feat - 2), jnp.bfloat16)
        laug_ref[...] = jnp.concatenate(
            [lz.astype(jnp.bfloat16), lh_hi, lh_lo, pad], axis=1)

    @pl.when(i + 2 < n)
    def _prefetch():
        pcopy(i + 2).start()

    pcopy(i).wait()
    x = pbuf[jax.lax.rem(i, NBUF)]
    uraw_ref[...] = -jnp.sum(jnp.exp(x) * x, axis=1)

    uq = u_ref[...]
    u_sq = jnp.sum(uq * uq, axis=1)
    uq_aug = jnp.concatenate(
        [uq.astype(jnp.bfloat16),
         jnp.ones((U_TILE, 2), jnp.bfloat16),
         jnp.zeros((U_TILE, K_AUG - n_feat - 2), jnp.bfloat16)], axis=1)

    carry = jnp.full((U_TILE, LANES), -jnp.inf, dtype=jnp.float32)
    for k in range(n_l // L_CHUNK):
        lc = laug_ref[pl.ds(k * L_CHUNK, L_CHUNK), :]
        t = jax.lax.dot_general(
            uq_aug, lc, (((1,), (1,)), ((), ())),
            preferred_element_type=jnp.float32)
        ms = [jnp.maximum(t[:, (2 * a) * LANES:(2 * a + 1) * LANES],
                          t[:, (2 * a + 1) * LANES:(2 * a + 2) * LANES])
              for a in range(L_CHUNK // (2 * LANES))]
        while len(ms) > 1:
            ms = [jnp.maximum(ms[b], ms[b + 1]) for b in range(0, len(ms), 2)]
        carry = jnp.maximum(carry, ms[0])
    d2min_ref[...] = u_sq - 2.0 * jnp.max(carry, axis=1)


def _finalize_kernel(uraw_ref, d2min_ref, lam_ref, out_ref):
    u = uraw_ref[...]
    u = u - jnp.min(u)
    u = u / (jnp.max(u) + 1e-18)
    d = jnp.sqrt(jnp.maximum(d2min_ref[...], 0.0))
    d = d - jnp.min(d)
    d = d / (jnp.max(d) + 1e-18)
    out_ref[...] = lam_ref[0] * u + d


@functools.partial(jax.jit, static_argnames=("interpret",))
def kernel(pred, U_z, L_z, lambda_, interpret=False):
    n_u = U_z.shape[0]
    n_l = L_z.shape[0]
    grid = (n_u // U_TILE,)
    uraw, d2min = pl.pallas_call(
        _main_kernel,
        grid=grid,
        in_specs=[
            pl.BlockSpec(memory_space=pl.ANY),
            pl.BlockSpec((U_TILE, U_z.shape[1]), lambda i: (i, 0)),
            pl.BlockSpec((n_l, L_z.shape[1]), lambda i: (0, 0)),
        ],
        out_specs=[
            pl.BlockSpec((U_TILE,), lambda i: (i,)),
            pl.BlockSpec((U_TILE,), lambda i: (i,)),
        ],
        out_shape=[
            jax.ShapeDtypeStruct((n_u,), jnp.float32),
            jax.ShapeDtypeStruct((n_u,), jnp.float32),
        ],
        scratch_shapes=[
            pltpu.VMEM((n_l, K_AUG), jnp.bfloat16),
            pltpu.VMEM((NBUF, U_TILE, pred.shape[1]), jnp.float32),
            pltpu.SemaphoreType.DMA((NBUF,)),
        ],
        interpret=interpret,
    )(pred, U_z, L_z)

    lam = jnp.asarray(lambda_, jnp.float32).reshape((1,))
    out = pl.pallas_call(
        _finalize_kernel,
        in_specs=[
            pl.BlockSpec((n_u,), lambda: (0,)),
            pl.BlockSpec((n_u,), lambda: (0,)),
            pl.BlockSpec(memory_space=pltpu.SMEM),
        ],
        out_shape=jax.ShapeDtypeStruct((n_u,), jnp.float32),
        interpret=interpret,
    )(uraw, d2min, lam)
    return out
